# Initial kernel scaffold; baseline (speedup 1.0000x reference)
import jax
import jax.numpy as jnp
from jax.experimental import pallas as pl


def kernel(x, edge_index, batch, W_gcn, b_gcn, W_ggc, W_ih, W_hh, b_ih, b_hh, W_lin1, b_lin1, W_lin2, b_lin2):
    raise NotImplementedError("write your pallas kernel here")


# speedup vs baseline: 16.0068x; 16.0068x over previous
import jax
import jax.numpy as jnp
from jax import lax
from jax.experimental import pallas as pl

PENALTY = 0.1
NEG_SLOPE = 0.01


def _lrelu(v):
    return jnp.where(v >= 0, v, NEG_SLOPE * v)


def _stnet_body(x_ref, src_ref, dst_ref, Wg_ref, bg_ref, Wggc_ref, Wih_ref,
                Whh_ref, bih_ref, bhh_ref, W1_ref, b1_ref, W2_ref, b2_ref,
                s_ref, min_ref, loss_ref):
    f32 = jnp.float32
    n = x_ref.shape[0]
    e = src_ref.shape[0]
    eb = 2048
    num_l = Wggc_ref.shape[0]
    h_dim = Wg_ref.shape[1]

    iota_col = lax.broadcasted_iota(jnp.int32, (n, 1), 0)
    iota_row = lax.broadcasted_iota(jnp.int32, (1, n), 1)

    A = jnp.zeros((n, n), f32)
    for b in range(e // eb):
        sblk = src_ref[pl.ds(b * eb, eb), :]
        dblk = dst_ref[:, pl.ds(b * eb, eb)]
        iota_e = lax.broadcasted_iota(jnp.int32, (eb, n), 1)
        Os = (sblk == iota_e).astype(jnp.bfloat16)
        OdT = (iota_col == dblk).astype(jnp.bfloat16)
        A = A + jnp.dot(OdT, Os, preferred_element_type=f32)

    rowsum = jnp.sum(A, axis=1, keepdims=True)
    deg = jnp.maximum(rowsum, 1.0)
    r = lax.rsqrt(deg)

    xw = jnp.dot(x_ref[...], Wg_ref[...], preferred_element_type=f32)
    agg = r * jnp.dot(A, r * xw, preferred_element_type=f32) + bg_ref[...]
    x1 = _lrelu(agg)

    h = x1
    for i in range(num_l):
        hw = jnp.dot(h, Wggc_ref[i], preferred_element_type=f32)
        m = jnp.dot(A, hw, preferred_element_type=f32)
        gi = jnp.dot(m, Wih_ref[...], preferred_element_type=f32) + bih_ref[...]
        gh = jnp.dot(h, Whh_ref[...], preferred_element_type=f32) + bhh_ref[...]
        rg = jax.nn.sigmoid(gi[:, :h_dim] + gh[:, :h_dim])
        z = jax.nn.sigmoid(gi[:, h_dim:2 * h_dim] + gh[:, h_dim:2 * h_dim])
        nn_ = jnp.tanh(gi[:, 2 * h_dim:] + rg * gh[:, 2 * h_dim:])
        h = (1.0 - z) * nn_ + z * h

    x2 = _lrelu(h) + x1
    x3 = _lrelu(jnp.dot(x2, W1_ref[...], preferred_element_type=f32) + b1_ref[...])
    xf_col = jax.nn.sigmoid(_lrelu(jnp.dot(x3, W2_ref[...],
                                           preferred_element_type=f32) + b2_ref[...]))

    eye = (iota_col == iota_row).astype(f32)
    xf_row = lax.dot_general(xf_col, eye, (((0,), (0,)), ((), ())),
                             preferred_element_type=f32)

    gt = (xf_col > xf_row).astype(f32)
    tie = jnp.logical_and(xf_col == xf_row, iota_col < iota_row).astype(f32)
    rank_row = jnp.sum(gt + tie, axis=0, keepdims=True)

    icolf = iota_col.astype(f32)
    M = (rank_row <= icolf).astype(f32)
    colsum = lax.dot_general(A, jnp.ones((n, 1), f32), (((0,), (0,)), ((), ())),
                             preferred_element_type=f32)
    rc = rowsum + colsum
    t12 = jnp.dot(M, rc, preferred_element_type=f32)
    Bm = jnp.dot(M, A, preferred_element_type=f32)
    t3 = jnp.sum(Bm * M, axis=1, keepdims=True)
    cut = t12 - 2.0 * t3
    f_unreg = -cut
    f_sets = f_unreg + PENALTY * (icolf + 1.0)

    s_ref[...] = xf_col
    min_ref[...] = jnp.min(f_unreg, axis=0, keepdims=True)
    loss_ref[...] = jnp.sum(f_sets, axis=0, keepdims=True) / n


def kernel(x, edge_index, batch, W_gcn, b_gcn, W_ggc, W_ih, W_hh, b_ih, b_hh,
           W_lin1, b_lin1, W_lin2, b_lin2):
    n = x.shape[0]
    e = edge_index.shape[1]
    f32 = jnp.float32
    src_col = edge_index[0].reshape(e, 1)
    dst_row = edge_index[1].reshape(1, e)
    s, mn, ls = pl.pallas_call(
        _stnet_body,
        out_shape=(
            jax.ShapeDtypeStruct((n, 1), f32),
            jax.ShapeDtypeStruct((1, 1), f32),
            jax.ShapeDtypeStruct((1, 1), f32),
        ),
    )(x, src_col, dst_row, W_gcn, b_gcn, W_ggc, W_ih, W_hh, b_ih, b_hh,
      W_lin1, b_lin1, W_lin2, b_lin2)
    return (s, mn.reshape(()), ls.reshape(()))
